# baseline (speedup 1.0000x reference)
import jax
import jax.numpy as jnp
from jax.experimental import pallas as pl


def _make_mlp_kernel(B, S, nb):
    def _mlp_kernel(x_ref, wr1_ref, bl1_ref, wr2_ref, bl2_ref, w_ref, out_ref):
        bl1 = bl1_ref[...].reshape(1, -1)
        bl2 = bl2_ref[...].reshape(1, -1)
        for i in range(nb):
            rows = min(S, B - i * S)
            h = jnp.dot(x_ref[i], wr1_ref[...],
                        preferred_element_type=jnp.float32)
            h = jnp.maximum(h + bl1, 0.0)
            h = jnp.dot(h, wr2_ref[...], preferred_element_type=jnp.float32)
            h = jnp.maximum(h + bl2, 0.0)
            s = jnp.dot(h, w_ref[...], preferred_element_type=jnp.float32)
            out_ref[pl.ds(i * S, rows), :] = s[:rows, :]
    return _mlp_kernel


def kernel(x0, Wl1, bl1, Wr1, Wl2, bl2, Wr2, weight, out_1, out_2):
    B, S, feat = x0.shape
    emb = Wr1.shape[1]
    nc = weight.shape[1]
    nb = -(-B // S)
    x_live = x0[:nb]

    return pl.pallas_call(
        _make_mlp_kernel(B, S, nb),
        grid=(1,),
        in_specs=[
            pl.BlockSpec((nb, S, feat), lambda i: (0, 0, 0)),
            pl.BlockSpec((feat, emb), lambda i: (0, 0)),
            pl.BlockSpec((emb,), lambda i: (0,)),
            pl.BlockSpec((emb, emb), lambda i: (0, 0)),
            pl.BlockSpec((emb,), lambda i: (0,)),
            pl.BlockSpec((emb, nc), lambda i: (0, 0)),
        ],
        out_specs=pl.BlockSpec((B, nc), lambda i: (0, 0)),
        out_shape=jax.ShapeDtypeStruct((B, nc), jnp.float32),
    )(x_live, Wr1, bl1, Wr2, bl2, weight)


# speedup vs baseline: 1.2199x; 1.2199x over previous
import jax
import jax.numpy as jnp
from jax.experimental import pallas as pl


def _make_mlp_kernel(B, S, nb):
    def _mlp_kernel(x_ref, wr1_ref, bl1_ref, wr2_ref, bl2_ref, w_ref, out_ref):
        bl1 = bl1_ref[...].reshape(1, -1)
        bl2 = bl2_ref[...].reshape(1, -1)
        for i in range(nb):
            rows = min(S, B - i * S)
            h = jnp.dot(x_ref[i], wr1_ref[...],
                        preferred_element_type=jnp.float32)
            h = jnp.maximum(h + bl1, 0.0)
            h = jnp.dot(h, wr2_ref[...], preferred_element_type=jnp.float32)
            h = jnp.maximum(h + bl2, 0.0)
            s = jax.lax.dot_general(h, w_ref[...], (((1,), (1,)), ((), ())),
                                    preferred_element_type=jnp.float32)
            out_ref[pl.ds(i * S, rows), :] = s[:rows, :]
    return _mlp_kernel


def kernel(x0, Wl1, bl1, Wr1, Wl2, bl2, Wr2, weight, out_1, out_2):
    B, S, feat = x0.shape
    emb = Wr1.shape[1]
    nc = weight.shape[1]
    nb = -(-B // S)
    x_live = x0[:nb]

    return pl.pallas_call(
        _make_mlp_kernel(B, S, nb),
        grid=(1,),
        in_specs=[
            pl.BlockSpec((nb, S, feat), lambda i: (0, 0, 0)),
            pl.BlockSpec((feat, emb), lambda i: (0, 0)),
            pl.BlockSpec((emb,), lambda i: (0,)),
            pl.BlockSpec((emb, emb), lambda i: (0, 0)),
            pl.BlockSpec((emb,), lambda i: (0,)),
            pl.BlockSpec((nc, emb), lambda i: (0, 0)),
        ],
        out_specs=pl.BlockSpec((B, nc), lambda i: (0, 0)),
        out_shape=jax.ShapeDtypeStruct((B, nc), jnp.float32),
    )(x_live, Wr1, bl1, Wr2, bl2, weight.T)


# speedup vs baseline: 2.8611x; 2.3453x over previous
import jax
import jax.numpy as jnp
from jax.experimental import pallas as pl

_SUB = 8


def _make_mlp_kernel(B, S, nb):
    def _mlp_kernel(x_ref, wr1_ref, bl1_ref, wr2_ref, bl2_ref, wt_ref,
                    out_ref):
        feat = x_ref.shape[-1]
        nc = wt_ref.shape[0]
        x = x_ref[...].reshape(S * _SUB, feat)
        h = jnp.dot(x, wr1_ref[...], preferred_element_type=jnp.float32)
        h = jnp.maximum(h + bl1_ref[...].reshape(1, -1), 0.0)
        h = jnp.dot(h, wr2_ref[...], preferred_element_type=jnp.float32)
        h = jnp.maximum(h + bl2_ref[...].reshape(1, -1), 0.0)
        s = jax.lax.dot_general(h, wt_ref[...], (((1,), (1,)), ((), ())),
                                preferred_element_type=jnp.float32)
        s3 = s.reshape(S, _SUB, nc)
        parts = []
        for b in range(nb):
            rows = min(S, B - b * S)
            parts.append(s3[:rows, b, :])
        out_ref[...] = jnp.concatenate(parts, axis=0).T
    return _mlp_kernel


def kernel(x0, Wl1, bl1, Wr1, Wl2, bl2, Wr2, weight, out_1, out_2):
    B, S, feat = x0.shape
    emb = Wr1.shape[1]
    nc = weight.shape[1]
    nb = -(-B // S)

    y = jnp.transpose(x0, (1, 0, 2))

    scores_t = pl.pallas_call(
        _make_mlp_kernel(B, S, nb),
        grid=(1,),
        in_specs=[
            pl.BlockSpec((S, _SUB, feat), lambda i: (0, 0, 0)),
            pl.BlockSpec((feat, emb), lambda i: (0, 0)),
            pl.BlockSpec((emb,), lambda i: (0,)),
            pl.BlockSpec((emb, emb), lambda i: (0, 0)),
            pl.BlockSpec((emb,), lambda i: (0,)),
            pl.BlockSpec((nc, emb), lambda i: (0, 0)),
        ],
        out_specs=pl.BlockSpec((nc, B), lambda i: (0, 0)),
        out_shape=jax.ShapeDtypeStruct((nc, B), jnp.float32),
    )(y, Wr1, bl1, Wr2, bl2, weight.T)
    return scores_t.T


# speedup vs baseline: 3.1041x; 1.0849x over previous
import jax
import jax.numpy as jnp
from jax.experimental import pallas as pl

_SUB = 8


def _make_mlp_kernel(B, S, nb):
    def _mlp_kernel(x_ref, wr1_ref, bl1_ref, wr2_ref, bl2_ref, wt_ref,
                    out_ref):
        feat = x_ref.shape[-1]
        nc = wt_ref.shape[0]
        parts = []
        for b in range(nb):
            rows = min(S, B - b * S)
            parts.append(x_ref[:rows, b, :])
        x = jnp.concatenate(parts, axis=0)
        h = jnp.dot(x, wr1_ref[...], preferred_element_type=jnp.float32)
        h = jnp.maximum(h + bl1_ref[...].reshape(1, -1), 0.0)
        h = jnp.dot(h, wr2_ref[...], preferred_element_type=jnp.float32)
        h = jnp.maximum(h + bl2_ref[...].reshape(1, -1), 0.0)
        s = jax.lax.dot_general(h, wt_ref[...], (((1,), (1,)), ((), ())),
                                preferred_element_type=jnp.float32)
        out_ref[...] = s.T
    return _mlp_kernel


def kernel(x0, Wl1, bl1, Wr1, Wl2, bl2, Wr2, weight, out_1, out_2):
    B, S, feat = x0.shape
    emb = Wr1.shape[1]
    nc = weight.shape[1]
    nb = -(-B // S)

    y = jnp.transpose(x0, (1, 0, 2))

    scores_t = pl.pallas_call(
        _make_mlp_kernel(B, S, nb),
        grid=(1,),
        in_specs=[
            pl.BlockSpec((S, _SUB, feat), lambda i: (0, 0, 0)),
            pl.BlockSpec((feat, emb), lambda i: (0, 0)),
            pl.BlockSpec((emb,), lambda i: (0,)),
            pl.BlockSpec((emb, emb), lambda i: (0, 0)),
            pl.BlockSpec((emb,), lambda i: (0,)),
            pl.BlockSpec((nc, emb), lambda i: (0, 0)),
        ],
        out_specs=pl.BlockSpec((nc, B), lambda i: (0, 0)),
        out_shape=jax.ShapeDtypeStruct((nc, B), jnp.float32),
    )(y, Wr1, bl1, Wr2, bl2, weight.T)
    return scores_t.T


# speedup vs baseline: 3.1605x; 1.0182x over previous
import jax
import jax.numpy as jnp
from jax.experimental import pallas as pl

_SUB = 8


def _make_mlp_kernel(B, S, nb):
    def _mlp_kernel(x_ref, wr1_ref, bl1_ref, wr2_ref, bl2_ref, wt_ref,
                    out_ref):
        feat = x_ref.shape[-1]
        nc = wt_ref.shape[0]
        parts = []
        for b in range(nb):
            rows = min(S, B - b * S)
            parts.append(x_ref[:rows, b, :])
        x = jnp.concatenate(parts, axis=0)
        h = jnp.dot(x, wr1_ref[...], preferred_element_type=jnp.float32)
        h = jnp.maximum(h + bl1_ref[...].reshape(1, -1), 0.0)
        h = jnp.dot(h, wr2_ref[...], preferred_element_type=jnp.float32)
        h = jnp.maximum(h + bl2_ref[...].reshape(1, -1), 0.0)
        out_ref[...] = jax.lax.dot_general(
            wt_ref[...], h, (((1,), (1,)), ((), ())),
            preferred_element_type=jnp.float32)
    return _mlp_kernel


def kernel(x0, Wl1, bl1, Wr1, Wl2, bl2, Wr2, weight, out_1, out_2):
    B, S, feat = x0.shape
    emb = Wr1.shape[1]
    nc = weight.shape[1]
    nb = -(-B // S)

    y = jnp.transpose(x0, (1, 0, 2))

    scores_t = pl.pallas_call(
        _make_mlp_kernel(B, S, nb),
        grid=(1,),
        in_specs=[
            pl.BlockSpec((S, _SUB, feat), lambda i: (0, 0, 0)),
            pl.BlockSpec((feat, emb), lambda i: (0, 0)),
            pl.BlockSpec((emb,), lambda i: (0,)),
            pl.BlockSpec((emb, emb), lambda i: (0, 0)),
            pl.BlockSpec((emb,), lambda i: (0,)),
            pl.BlockSpec((nc, emb), lambda i: (0, 0)),
        ],
        out_specs=pl.BlockSpec((nc, B), lambda i: (0, 0)),
        out_shape=jax.ShapeDtypeStruct((nc, B), jnp.float32),
    )(y, Wr1, bl1, Wr2, bl2, weight.T)
    return scores_t.T
